# baseline (speedup 1.0000x reference)
import jax
import jax.numpy as jnp
from jax.experimental import pallas as pl
from jax.experimental.pallas import tpu as pltpu

_NUM_JOINTS = 25
_NUM_ACTORS = 8
_NUM_CLASSES = 6
_FEAT = 2048

_BB = 8
_BP = 48
_OFF = 8
_SS3 = 496
_SS5 = 288
_R0 = 48
_NR3 = _R0 + (_BB - 1) * _SS3 + 560
_NR5 = _R0 + (_BB - 1) * _SS5 + 352


def _slab(a, c):
    f32 = jnp.float32
    z1 = jnp.zeros((1, c), f32)
    z2 = jnp.zeros((2, c), f32)
    return jnp.concatenate([
        jnp.concatenate([z2, a], axis=0),
        jnp.concatenate([z1, a, z1], axis=0),
        jnp.concatenate([a, z2], axis=0)], axis=1)


def _conv_kernel(xr_ref, wfm_ref, b2t_ref, w3s_ref, b3_ref, w4s_ref, b4_ref,
                 w5s_ref, b5_ref, se16e_ref, se16o_ref, se8e_ref, se8o_ref,
                 out_ref, buf3, buf4, buf5):
    f32 = jnp.float32

    for buf, ss, blo, bhi in ((buf3, _SS3, 424, 552), (buf4, _SS3, 424, 552),
                              (buf5, _SS5, 240, 344)):
        buf[0:_R0 + 56, :] = jnp.zeros((_R0 + 56, buf.shape[1]), f32)
        for s in range(_BB):
            r = _R0 + s * ss
            buf[r + blo: r + bhi, :] = jnp.zeros((bhi - blo, buf.shape[1]), f32)

    pa = jnp.dot(wfm_ref[...], xr_ref[0], preferred_element_type=f32)
    pa = pa + b2t_ref[...]
    for s in range(_BB):
        base = _R0 + s * _SS3
        for w in range(_NUM_ACTORS):
            a = pa[:, (s * 8 + w) * 32: (s * 8 + w) * 32 + 32]
            r0 = base + (w + 1) * _BP + _OFF
            buf3[r0 - 1: r0 + 33, :] = _slab(a, 32)

    for s in range(_BB):
        base = _R0 + s * _SS3
        p3 = jnp.dot(buf3[base + 8: base + 488, :], w3s_ref[...],
                     preferred_element_type=f32)
        y3 = (p3[0:384, 0:64] + p3[48:432, 64:128] + p3[96:480, 128:192]
              + b3_ref[...])
        for w in range(_NUM_ACTORS):
            a3 = y3[48 * w: 48 * w + 32, :]
            r0 = base + (w + 1) * _BP + _OFF
            buf4[r0 - 1: r0 + 33, :] = _slab(a3, 64)

    mcat = []
    for s in range(_BB):
        base = _R0 + s * _SS3
        p4 = jnp.dot(buf4[base + 8: base + 488, :], w4s_ref[...],
                     preferred_element_type=f32)
        y4 = (p4[0:384, 0:32] + p4[48:432, 32:64] + p4[96:480, 64:96]
              + b4_ref[...])
        for a2 in range(4):
            mcat.append(jnp.maximum(y4[96 * a2: 96 * a2 + 32, :],
                                    y4[96 * a2 + 48: 96 * a2 + 80, :]))
    mcat = jnp.concatenate(mcat, axis=1)
    p4a = jnp.maximum(
        jnp.dot(se16e_ref[...], mcat, preferred_element_type=f32),
        jnp.dot(se16o_ref[...], mcat, preferred_element_type=f32))
    for s in range(_BB):
        base5 = _R0 + s * _SS5
        for a2 in range(4):
            c0 = (4 * s + a2) * 32
            p4 = p4a[:, c0: c0 + 32]
            r0 = base5 + (a2 + 1) * _BP + _OFF
            buf5[r0 - 1: r0 + 17, :] = _slab(p4, 32)

    m5cat = []
    for s in range(_BB):
        base5 = _R0 + s * _SS5
        p5 = jnp.dot(buf5[base5 + 8: base5 + 264, :], w5s_ref[...],
                     preferred_element_type=f32)
        y5 = (p5[0:160, 0:128] + p5[48:208, 128:256] + p5[96:256, 256:384]
              + b5_ref[...])
        for w2 in range(2):
            m5cat.append(jnp.maximum(y5[96 * w2: 96 * w2 + 16, :],
                                     y5[96 * w2 + 48: 96 * w2 + 64, :]))
    m5cat = jnp.concatenate(m5cat, axis=1)
    p5a = jnp.maximum(
        jnp.dot(se8e_ref[...], m5cat, preferred_element_type=f32),
        jnp.dot(se8o_ref[...], m5cat, preferred_element_type=f32))
    p5a = jnp.maximum(p5a, 0.0)
    for s in range(_BB):
        for w2 in range(2):
            c0 = (2 * s + w2) * 128
            out_ref[s, w2 * 8: w2 * 8 + 8, :] = p5a[:, c0: c0 + 128]


def _fc_head_kernel(x_ref, w1_ref, b1_ref, w2_ref, b2_ref, o_ref):
    h = jnp.dot(x_ref[...], w1_ref[...], preferred_element_type=jnp.float32)
    h = h + b1_ref[...]
    y = jnp.dot(h, w2_ref[...], preferred_element_type=jnp.float32)
    o_ref[...] = y + b2_ref[...]


def _conv_features(xr, wfm, b2t, w3s, b3, w4s, b4, w5s, b5,
                   se16e, se16o, se8e, se8o):
    nb = xr.shape[0]
    return pl.pallas_call(
        _conv_kernel,
        out_shape=jax.ShapeDtypeStruct((nb * _BB, 16, 128), jnp.float32),
        grid=(nb,),
        in_specs=[
            pl.BlockSpec((1, 8, _BB * 256), lambda i: (i, 0, 0)),
            pl.BlockSpec((32, 8), lambda i: (0, 0)),
            pl.BlockSpec((32, _BB * 256), lambda i: (0, 0)),
            pl.BlockSpec((96, 192), lambda i: (0, 0)),
            pl.BlockSpec((1, 64), lambda i: (0, 0)),
            pl.BlockSpec((192, 96), lambda i: (0, 0)),
            pl.BlockSpec((1, 32), lambda i: (0, 0)),
            pl.BlockSpec((96, 384), lambda i: (0, 0)),
            pl.BlockSpec((1, 128), lambda i: (0, 0)),
            pl.BlockSpec((16, 32), lambda i: (0, 0)),
            pl.BlockSpec((16, 32), lambda i: (0, 0)),
            pl.BlockSpec((8, 16), lambda i: (0, 0)),
            pl.BlockSpec((8, 16), lambda i: (0, 0)),
        ],
        out_specs=pl.BlockSpec((_BB, 16, 128), lambda i: (i, 0, 0)),
        scratch_shapes=[
            pltpu.VMEM((_NR3, 96), jnp.float32),
            pltpu.VMEM((_NR3, 192), jnp.float32),
            pltpu.VMEM((_NR5, 96), jnp.float32),
        ],
        compiler_params=pltpu.CompilerParams(dimension_semantics=("parallel",)),
    )(xr, wfm, b2t, w3s, b3, w4s, b4, w5s, b5, se16e, se16o, se8e, se8o)


def _fc_head(person, w1t, b1f, w2p, b2f):
    Bp = person.shape[0]
    bm = next(d for d in (256, 128, 64, 32, 16, 8) if Bp % d == 0)
    return pl.pallas_call(
        _fc_head_kernel,
        out_shape=jax.ShapeDtypeStruct((Bp, 128), jnp.float32),
        grid=(Bp // bm,),
        in_specs=[
            pl.BlockSpec((bm, _FEAT), lambda i: (i, 0)),
            pl.BlockSpec((_FEAT, 256), lambda i: (0, 0)),
            pl.BlockSpec((1, 256), lambda i: (0, 0)),
            pl.BlockSpec((256, 128), lambda i: (0, 0)),
            pl.BlockSpec((1, 128), lambda i: (0, 0)),
        ],
        out_specs=pl.BlockSpec((bm, 128), lambda i: (i, 0)),
        compiler_params=pltpu.CompilerParams(dimension_semantics=("parallel",)),
    )(person, w1t, b1f, w2p, b2f)


@jax.jit
def _forward(X, wfa, b2m, w3, b3, w4, b4, w5, b5,
             se16e, se16o, se8e, se8o, w1t, b1f, w2p, b2f):
    f32 = jnp.float32
    x = X.reshape(-1, 2, _NUM_JOINTS, _NUM_ACTORS).astype(f32)
    B = x.shape[0]
    Bp = ((B + _BB - 1) // _BB) * _BB
    nb = Bp // _BB

    xpad = jnp.pad(x, ((0, Bp - B), (0, 0), (1, 8), (0, 0)))
    taps = [xpad[:, kind, kh: kh + 32, :].transpose(0, 2, 1)
            for kind in range(2) for kh in range(3)]
    xr = jnp.stack(taps, axis=1)
    xr = xr.reshape(nb, _BB, 6, 256).transpose(0, 2, 1, 3).reshape(nb, 6, _BB * 256)
    xr = jnp.pad(xr, ((0, 0), (0, 2), (0, 0)))

    wfm = jnp.pad(jnp.transpose(wfa[..., 0], (2, 1, 0)).reshape(32, 6),
                  ((0, 0), (0, 2)))
    b2t = jnp.tile(b2m, (1, _BB * 8))
    w3s = jnp.transpose(w3, (1, 0, 2)).reshape(96, 192)
    w4s = jnp.transpose(w4, (1, 0, 2)).reshape(192, 96)
    w5s = jnp.transpose(w5, (1, 0, 2)).reshape(96, 384)

    feats = _conv_features(xr, wfm, b2t, w3s, b3, w4s, b4, w5s, b5,
                           se16e, se16o, se8e, se8o)
    person = feats.reshape(Bp, _FEAT)
    out = _fc_head(person, w1t, b1f, w2p, b2f)
    return out[:B, :_NUM_CLASSES]


def kernel(X, wfa, b2m, w3, b3, w4, b4, w5, b5,
           se16e, se16o, se8e, se8o, w1t, b1f, w2p, b2f):
    return _forward(X, wfa, b2m, w3, b3, w4, b4, w5, b5,
                    se16e, se16o, se8e, se8o, w1t, b1f, w2p, b2f)


# speedup vs baseline: 1.4948x; 1.4948x over previous
import jax
import jax.numpy as jnp
from jax.experimental import pallas as pl
from jax.experimental.pallas import tpu as pltpu

_NUM_JOINTS = 25
_NUM_ACTORS = 8
_NUM_CLASSES = 6
_FEAT = 2048

_BB = 8
_BP = 48
_OFF = 8
_SS3 = 496
_SS5 = 288
_R0 = 48
_NR3 = _R0 + (_BB - 1) * _SS3 + 560
_NR5 = _R0 + (_BB - 1) * _SS5 + 352


def _slab(a, c):
    f32 = jnp.float32
    z1 = jnp.zeros((1, c), f32)
    z2 = jnp.zeros((2, c), f32)
    return jnp.concatenate([
        jnp.concatenate([z2, a], axis=0),
        jnp.concatenate([z1, a, z1], axis=0),
        jnp.concatenate([a, z2], axis=0)], axis=1)


def _conv_kernel(xr_ref, wfm_ref, b2t_ref, w3s_ref, b3_ref, w4s_ref, b4_ref,
                 w5s_ref, b5_ref, se16e_ref, se16o_ref, se8e_ref, se8o_ref,
                 out_ref, buf3, buf4, buf5):
    f32 = jnp.float32

    for buf, ss, blo, bhi in ((buf3, _SS3, 424, 552), (buf4, _SS3, 424, 552),
                              (buf5, _SS5, 240, 344)):
        buf[0:_R0 + 56, :] = jnp.zeros((_R0 + 56, buf.shape[1]), f32)
        for s in range(_BB):
            r = _R0 + s * ss
            buf[r + blo: r + bhi, :] = jnp.zeros((bhi - blo, buf.shape[1]), f32)

    pa = jnp.dot(wfm_ref[...], xr_ref[0], preferred_element_type=f32)
    pa = pa + b2t_ref[...]
    for s in range(_BB):
        base = _R0 + s * _SS3
        for w in range(_NUM_ACTORS):
            a = pa[:, (s * 8 + w) * 32: (s * 8 + w) * 32 + 32]
            r0 = base + (w + 1) * _BP + _OFF
            buf3[r0 - 1: r0 + 33, :] = _slab(a, 32)

    for s in range(_BB):
        base = _R0 + s * _SS3
        y3 = (jnp.dot(buf3[base + 8: base + 392, :], w3s_ref[0],
                      preferred_element_type=f32)
              + jnp.dot(buf3[base + 56: base + 440, :], w3s_ref[1],
                        preferred_element_type=f32)
              + jnp.dot(buf3[base + 104: base + 488, :], w3s_ref[2],
                        preferred_element_type=f32)
              + b3_ref[...])
        for w in range(_NUM_ACTORS):
            a3 = y3[48 * w: 48 * w + 32, :]
            r0 = base + (w + 1) * _BP + _OFF
            buf4[r0 - 1: r0 + 33, :] = _slab(a3, 64)

    mcat = []
    for s in range(_BB):
        base = _R0 + s * _SS3
        y4 = (jnp.dot(buf4[base + 8: base + 392, :], w4s_ref[0],
                      preferred_element_type=f32)
              + jnp.dot(buf4[base + 56: base + 440, :], w4s_ref[1],
                        preferred_element_type=f32)
              + jnp.dot(buf4[base + 104: base + 488, :], w4s_ref[2],
                        preferred_element_type=f32)
              + b4_ref[...])
        for a2 in range(4):
            mcat.append(jnp.maximum(y4[96 * a2: 96 * a2 + 32, :],
                                    y4[96 * a2 + 48: 96 * a2 + 80, :]))
    mcat = jnp.concatenate(mcat, axis=1)
    p4a = jnp.maximum(
        jnp.dot(se16e_ref[...], mcat, preferred_element_type=f32),
        jnp.dot(se16o_ref[...], mcat, preferred_element_type=f32))
    for s in range(_BB):
        base5 = _R0 + s * _SS5
        for a2 in range(4):
            c0 = (4 * s + a2) * 32
            p4 = p4a[:, c0: c0 + 32]
            r0 = base5 + (a2 + 1) * _BP + _OFF
            buf5[r0 - 1: r0 + 17, :] = _slab(p4, 32)

    m5cat = []
    for s in range(_BB):
        base5 = _R0 + s * _SS5
        y5 = (jnp.dot(buf5[base5 + 8: base5 + 176, :], w5s_ref[0],
                      preferred_element_type=f32)
              + jnp.dot(buf5[base5 + 56: base5 + 224, :], w5s_ref[1],
                        preferred_element_type=f32)
              + jnp.dot(buf5[base5 + 104: base5 + 272, :], w5s_ref[2],
                        preferred_element_type=f32)
              + b5_ref[...])
        for w2 in range(2):
            m5cat.append(jnp.maximum(y5[96 * w2: 96 * w2 + 16, :],
                                     y5[96 * w2 + 48: 96 * w2 + 64, :]))
    m5cat = jnp.concatenate(m5cat, axis=1)
    p5a = jnp.maximum(
        jnp.dot(se8e_ref[...], m5cat, preferred_element_type=f32),
        jnp.dot(se8o_ref[...], m5cat, preferred_element_type=f32))
    p5a = jnp.maximum(p5a, 0.0)
    for s in range(_BB):
        for w2 in range(2):
            c0 = (2 * s + w2) * 128
            out_ref[s, w2 * 8: w2 * 8 + 8, :] = p5a[:, c0: c0 + 128]


def _fc_head_kernel(x_ref, w1_ref, b1_ref, w2_ref, b2_ref, o_ref):
    h = jnp.dot(x_ref[...], w1_ref[...], preferred_element_type=jnp.float32)
    h = h + b1_ref[...]
    y = jnp.dot(h, w2_ref[...], preferred_element_type=jnp.float32)
    o_ref[...] = y + b2_ref[...]


def _conv_features(xr, wfm, b2t, w3s, b3, w4s, b4, w5s, b5,
                   se16e, se16o, se8e, se8o):
    nb = xr.shape[0]
    return pl.pallas_call(
        _conv_kernel,
        out_shape=jax.ShapeDtypeStruct((nb * _BB, 16, 128), jnp.float32),
        grid=(nb,),
        in_specs=[
            pl.BlockSpec((1, 8, _BB * 256), lambda i: (i, 0, 0)),
            pl.BlockSpec((32, 8), lambda i: (0, 0)),
            pl.BlockSpec((32, _BB * 256), lambda i: (0, 0)),
            pl.BlockSpec((3, 96, 64), lambda i: (0, 0, 0)),
            pl.BlockSpec((1, 64), lambda i: (0, 0)),
            pl.BlockSpec((3, 192, 32), lambda i: (0, 0, 0)),
            pl.BlockSpec((1, 32), lambda i: (0, 0)),
            pl.BlockSpec((3, 96, 128), lambda i: (0, 0, 0)),
            pl.BlockSpec((1, 128), lambda i: (0, 0)),
            pl.BlockSpec((16, 32), lambda i: (0, 0)),
            pl.BlockSpec((16, 32), lambda i: (0, 0)),
            pl.BlockSpec((8, 16), lambda i: (0, 0)),
            pl.BlockSpec((8, 16), lambda i: (0, 0)),
        ],
        out_specs=pl.BlockSpec((_BB, 16, 128), lambda i: (i, 0, 0)),
        scratch_shapes=[
            pltpu.VMEM((_NR3, 96), jnp.float32),
            pltpu.VMEM((_NR3, 192), jnp.float32),
            pltpu.VMEM((_NR5, 96), jnp.float32),
        ],
        compiler_params=pltpu.CompilerParams(dimension_semantics=("parallel",)),
    )(xr, wfm, b2t, w3s, b3, w4s, b4, w5s, b5, se16e, se16o, se8e, se8o)


def _fc_head(person, w1t, b1f, w2p, b2f):
    Bp = person.shape[0]
    bm = next(d for d in (256, 128, 64, 32, 16, 8) if Bp % d == 0)
    return pl.pallas_call(
        _fc_head_kernel,
        out_shape=jax.ShapeDtypeStruct((Bp, 128), jnp.float32),
        grid=(Bp // bm,),
        in_specs=[
            pl.BlockSpec((bm, _FEAT), lambda i: (i, 0)),
            pl.BlockSpec((_FEAT, 256), lambda i: (0, 0)),
            pl.BlockSpec((1, 256), lambda i: (0, 0)),
            pl.BlockSpec((256, 128), lambda i: (0, 0)),
            pl.BlockSpec((1, 128), lambda i: (0, 0)),
        ],
        out_specs=pl.BlockSpec((bm, 128), lambda i: (i, 0)),
        compiler_params=pltpu.CompilerParams(dimension_semantics=("parallel",)),
    )(person, w1t, b1f, w2p, b2f)


@jax.jit
def _forward(X, wfa, b2m, w3, b3, w4, b4, w5, b5,
             se16e, se16o, se8e, se8o, w1t, b1f, w2p, b2f):
    f32 = jnp.float32
    x = X.reshape(-1, 2, _NUM_JOINTS, _NUM_ACTORS).astype(f32)
    B = x.shape[0]
    Bp = ((B + _BB - 1) // _BB) * _BB
    nb = Bp // _BB

    xpad = jnp.pad(x, ((0, Bp - B), (0, 0), (1, 8), (0, 0)))
    taps = [xpad[:, kind, kh: kh + 32, :].transpose(0, 2, 1)
            for kind in range(2) for kh in range(3)]
    xr = jnp.stack(taps, axis=1)
    xr = xr.reshape(nb, _BB, 6, 256).transpose(0, 2, 1, 3).reshape(nb, 6, _BB * 256)
    xr = jnp.pad(xr, ((0, 0), (0, 2), (0, 0)))

    wfm = jnp.pad(jnp.transpose(wfa[..., 0], (2, 1, 0)).reshape(32, 6),
                  ((0, 0), (0, 2)))
    b2t = jnp.tile(b2m, (1, _BB * 8))
    w3s, w4s, w5s = w3, w4, w5

    feats = _conv_features(xr, wfm, b2t, w3s, b3, w4s, b4, w5s, b5,
                           se16e, se16o, se8e, se8o)
    person = feats.reshape(Bp, _FEAT)
    out = _fc_head(person, w1t, b1f, w2p, b2f)
    return out[:B, :_NUM_CLASSES]


def kernel(X, wfa, b2m, w3, b3, w4, b4, w5, b5,
           se16e, se16o, se8e, se8o, w1t, b1f, w2p, b2f):
    return _forward(X, wfa, b2m, w3, b3, w4, b4, w5, b5,
                    se16e, se16o, se8e, se8o, w1t, b1f, w2p, b2f)


# speedup vs baseline: 1.6787x; 1.1230x over previous
import jax
import jax.numpy as jnp
from jax.experimental import pallas as pl
from jax.experimental.pallas import tpu as pltpu

_NUM_JOINTS = 25
_NUM_ACTORS = 8
_NUM_CLASSES = 6
_FEAT = 2048

_BB = 8
_BP = 48
_OFF = 8
_SS3 = 496
_SS5 = 288
_R0 = 48
_NR3 = _R0 + (_BB - 1) * _SS3 + 560
_NR5 = _R0 + (_BB - 1) * _SS5 + 352


def _slab(a, c):
    z1 = jnp.zeros((1, c), a.dtype)
    z2 = jnp.zeros((2, c), a.dtype)
    return jnp.concatenate([
        jnp.concatenate([z2, a], axis=0),
        jnp.concatenate([z1, a, z1], axis=0),
        jnp.concatenate([a, z2], axis=0)], axis=1)


def _conv_kernel(xr_ref, wfm_ref, b2t_ref, w3s_ref, b3_ref, w4s_ref, b4_ref,
                 w5s_ref, b5_ref, se16e_ref, se16o_ref, se8e_ref, se8o_ref,
                 out_ref, buf3, buf4, buf5):
    f32 = jnp.float32

    for buf, ss, blo, bhi in ((buf3, _SS3, 424, 552), (buf4, _SS3, 424, 552),
                              (buf5, _SS5, 240, 344)):
        buf[0:_R0 + 56, :] = jnp.zeros((_R0 + 56, buf.shape[1]), buf.dtype)
        for s in range(_BB):
            r = _R0 + s * ss
            buf[r + blo: r + bhi, :] = jnp.zeros((bhi - blo, buf.shape[1]), buf.dtype)

    pa = jnp.dot(wfm_ref[...], xr_ref[0], preferred_element_type=f32)
    pa = pa + b2t_ref[...]
    pa = pa.astype(jnp.bfloat16)
    for s in range(_BB):
        base = _R0 + s * _SS3
        for w in range(_NUM_ACTORS):
            a = pa[:, (s * 8 + w) * 32: (s * 8 + w) * 32 + 32]
            r0 = base + (w + 1) * _BP + _OFF
            buf3[r0 - 1: r0 + 33, :] = _slab(a, 32)

    for s in range(_BB):
        base = _R0 + s * _SS3
        y3 = (jnp.dot(buf3[base + 8: base + 392, :], w3s_ref[0],
                      preferred_element_type=f32)
              + jnp.dot(buf3[base + 56: base + 440, :], w3s_ref[1],
                        preferred_element_type=f32)
              + jnp.dot(buf3[base + 104: base + 488, :], w3s_ref[2],
                        preferred_element_type=f32)
              + b3_ref[...]).astype(jnp.bfloat16)
        for w in range(_NUM_ACTORS):
            a3 = y3[48 * w: 48 * w + 32, :]
            r0 = base + (w + 1) * _BP + _OFF
            buf4[r0 - 1: r0 + 33, :] = _slab(a3, 64)

    mcat = []
    for s in range(_BB):
        base = _R0 + s * _SS3
        y4 = (jnp.dot(buf4[base + 8: base + 392, :], w4s_ref[0],
                      preferred_element_type=f32)
              + jnp.dot(buf4[base + 56: base + 440, :], w4s_ref[1],
                        preferred_element_type=f32)
              + jnp.dot(buf4[base + 104: base + 488, :], w4s_ref[2],
                        preferred_element_type=f32)
              + b4_ref[...])
        for a2 in range(4):
            mcat.append(jnp.maximum(y4[96 * a2: 96 * a2 + 32, :],
                                    y4[96 * a2 + 48: 96 * a2 + 80, :]))
    mcat = jnp.concatenate(mcat, axis=1)
    p4a = jnp.maximum(
        jnp.dot(se16e_ref[...], mcat, preferred_element_type=f32),
        jnp.dot(se16o_ref[...], mcat, preferred_element_type=f32))
    p4a = p4a.astype(jnp.bfloat16)
    for s in range(_BB):
        base5 = _R0 + s * _SS5
        for a2 in range(4):
            c0 = (4 * s + a2) * 32
            p4 = p4a[:, c0: c0 + 32]
            r0 = base5 + (a2 + 1) * _BP + _OFF
            buf5[r0 - 1: r0 + 17, :] = _slab(p4, 32)

    m5cat = []
    for s in range(_BB):
        base5 = _R0 + s * _SS5
        y5 = (jnp.dot(buf5[base5 + 8: base5 + 176, :], w5s_ref[0],
                      preferred_element_type=f32)
              + jnp.dot(buf5[base5 + 56: base5 + 224, :], w5s_ref[1],
                        preferred_element_type=f32)
              + jnp.dot(buf5[base5 + 104: base5 + 272, :], w5s_ref[2],
                        preferred_element_type=f32)
              + b5_ref[...])
        for w2 in range(2):
            m5cat.append(jnp.maximum(y5[96 * w2: 96 * w2 + 16, :],
                                     y5[96 * w2 + 48: 96 * w2 + 64, :]))
    m5cat = jnp.concatenate(m5cat, axis=1)
    p5a = jnp.maximum(
        jnp.dot(se8e_ref[...], m5cat, preferred_element_type=f32),
        jnp.dot(se8o_ref[...], m5cat, preferred_element_type=f32))
    p5a = jnp.maximum(p5a, 0.0)
    for s in range(_BB):
        for w2 in range(2):
            c0 = (2 * s + w2) * 128
            out_ref[s, w2 * 8: w2 * 8 + 8, :] = p5a[:, c0: c0 + 128]


def _fc_head_kernel(x_ref, w1_ref, b1_ref, w2_ref, b2_ref, o_ref):
    h = jnp.dot(x_ref[...], w1_ref[...], preferred_element_type=jnp.float32)
    h = h + b1_ref[...]
    y = jnp.dot(h, w2_ref[...], preferred_element_type=jnp.float32)
    o_ref[...] = y + b2_ref[...]


def _conv_features(xr, wfm, b2t, w3s, b3, w4s, b4, w5s, b5,
                   se16e, se16o, se8e, se8o):
    nb = xr.shape[0]
    return pl.pallas_call(
        _conv_kernel,
        out_shape=jax.ShapeDtypeStruct((nb * _BB, 16, 128), jnp.float32),
        grid=(nb,),
        in_specs=[
            pl.BlockSpec((1, 8, _BB * 256), lambda i: (i, 0, 0)),
            pl.BlockSpec((32, 8), lambda i: (0, 0)),
            pl.BlockSpec((32, _BB * 256), lambda i: (0, 0)),
            pl.BlockSpec((3, 96, 64), lambda i: (0, 0, 0)),
            pl.BlockSpec((1, 64), lambda i: (0, 0)),
            pl.BlockSpec((3, 192, 32), lambda i: (0, 0, 0)),
            pl.BlockSpec((1, 32), lambda i: (0, 0)),
            pl.BlockSpec((3, 96, 128), lambda i: (0, 0, 0)),
            pl.BlockSpec((1, 128), lambda i: (0, 0)),
            pl.BlockSpec((16, 32), lambda i: (0, 0)),
            pl.BlockSpec((16, 32), lambda i: (0, 0)),
            pl.BlockSpec((8, 16), lambda i: (0, 0)),
            pl.BlockSpec((8, 16), lambda i: (0, 0)),
        ],
        out_specs=pl.BlockSpec((_BB, 16, 128), lambda i: (i, 0, 0)),
        scratch_shapes=[
            pltpu.VMEM((_NR3, 96), jnp.bfloat16),
            pltpu.VMEM((_NR3, 192), jnp.bfloat16),
            pltpu.VMEM((_NR5, 96), jnp.bfloat16),
        ],
        compiler_params=pltpu.CompilerParams(dimension_semantics=("parallel",)),
    )(xr, wfm, b2t, w3s, b3, w4s, b4, w5s, b5, se16e, se16o, se8e, se8o)


def _fc_head(person, w1t, b1f, w2p, b2f):
    Bp = person.shape[0]
    bm = next(d for d in (256, 128, 64, 32, 16, 8) if Bp % d == 0)
    return pl.pallas_call(
        _fc_head_kernel,
        out_shape=jax.ShapeDtypeStruct((Bp, 128), jnp.float32),
        grid=(Bp // bm,),
        in_specs=[
            pl.BlockSpec((bm, _FEAT), lambda i: (i, 0)),
            pl.BlockSpec((_FEAT, 256), lambda i: (0, 0)),
            pl.BlockSpec((1, 256), lambda i: (0, 0)),
            pl.BlockSpec((256, 128), lambda i: (0, 0)),
            pl.BlockSpec((1, 128), lambda i: (0, 0)),
        ],
        out_specs=pl.BlockSpec((bm, 128), lambda i: (i, 0)),
        compiler_params=pltpu.CompilerParams(dimension_semantics=("parallel",)),
    )(person, w1t, b1f, w2p, b2f)


@jax.jit
def _forward(X, wfa, b2m, w3, b3, w4, b4, w5, b5,
             se16e, se16o, se8e, se8o, w1t, b1f, w2p, b2f):
    f32 = jnp.float32
    x = X.reshape(-1, 2, _NUM_JOINTS, _NUM_ACTORS).astype(f32)
    B = x.shape[0]
    Bp = ((B + _BB - 1) // _BB) * _BB
    nb = Bp // _BB

    xpad = jnp.pad(x, ((0, Bp - B), (0, 0), (1, 8), (0, 0)))
    taps = [xpad[:, kind, kh: kh + 32, :].transpose(0, 2, 1)
            for kind in range(2) for kh in range(3)]
    xr = jnp.stack(taps, axis=1)
    xr = xr.reshape(nb, _BB, 6, 256).transpose(0, 2, 1, 3).reshape(nb, 6, _BB * 256)
    xr = jnp.pad(xr, ((0, 0), (0, 2), (0, 0))).astype(jnp.bfloat16)

    wfm = jnp.pad(jnp.transpose(wfa[..., 0], (2, 1, 0)).reshape(32, 6),
                  ((0, 0), (0, 2))).astype(jnp.bfloat16)
    b2t = jnp.tile(b2m, (1, _BB * 8))
    w3s = w3.astype(jnp.bfloat16)
    w4s = w4.astype(jnp.bfloat16)
    w5s = w5.astype(jnp.bfloat16)

    feats = _conv_features(xr, wfm, b2t, w3s, b3, w4s, b4, w5s, b5,
                           se16e, se16o, se8e, se8o)
    person = feats.reshape(Bp, _FEAT)
    out = _fc_head(person, w1t, b1f, w2p, b2f)
    return out[:B, :_NUM_CLASSES]


def kernel(X, wfa, b2m, w3, b3, w4, b4, w5, b5,
           se16e, se16o, se8e, se8o, w1t, b1f, w2p, b2f):
    return _forward(X, wfa, b2m, w3, b3, w4, b4, w5, b5,
                    se16e, se16o, se8e, se8o, w1t, b1f, w2p, b2f)


# speedup vs baseline: 3.4838x; 2.0754x over previous
import jax
import jax.numpy as jnp
from jax.experimental import pallas as pl
from jax.experimental.pallas import tpu as pltpu

_NUM_JOINTS = 25
_NUM_ACTORS = 8
_NUM_CLASSES = 6
_FEAT = 2048

_BB = 8
_SH = 40
_SH5 = 24
_H0 = 8
_NR = _H0 + _BB * _SH + 8
_NR5 = _H0 + _BB * _SH5 + 8


def _conv_kernel(xr_ref, wfm_ref, b2t_ref, w3b_ref, b3t_ref, w4b_ref, b4t_ref,
                 w5b_ref, b5t_ref, se16e_ref, se16o_ref, se8e_ref, se8o_ref,
                 out_ref, buf3, buf4, buf5):
    f32 = jnp.float32
    bf16 = jnp.bfloat16

    for buf, ss, nv in ((buf3, _SH, 32), (buf4, _SH, 32), (buf5, _SH5, 16)):
        buf[0:_H0, :] = jnp.zeros((_H0, buf.shape[1]), bf16)
        for s in range(_BB):
            r = _H0 + s * ss + nv
            buf[r: r + 8, :] = jnp.zeros((8, buf.shape[1]), bf16)

    pa = jnp.dot(wfm_ref[...], xr_ref[0], preferred_element_type=f32)
    pa = (pa + b2t_ref[...]).astype(bf16)
    for s in range(_BB):
        buf3[_H0 + s * _SH: _H0 + s * _SH + 32, :] = pa[:, s * 256: s * 256 + 256]

    for c in range(2):
        lo = _H0 + c * 4 * _SH
        m = 4 * _SH - 8
        y3 = (jnp.dot(buf3[lo - 1: lo - 1 + m, :], w3b_ref[0],
                      preferred_element_type=f32)
              + jnp.dot(buf3[lo: lo + m, :], w3b_ref[1],
                        preferred_element_type=f32)
              + jnp.dot(buf3[lo + 1: lo + 1 + m, :], w3b_ref[2],
                        preferred_element_type=f32)
              + b3t_ref[...]).astype(bf16)
        for s in range(4):
            buf4[lo + s * _SH: lo + s * _SH + 32, :] = y3[s * _SH: s * _SH + 32, :]

    mcat = []
    for c in range(2):
        lo = _H0 + c * 4 * _SH
        m = 4 * _SH - 8
        y4 = (jnp.dot(buf4[lo - 1: lo - 1 + m, :], w4b_ref[0],
                      preferred_element_type=f32)
              + jnp.dot(buf4[lo: lo + m, :], w4b_ref[1],
                        preferred_element_type=f32)
              + jnp.dot(buf4[lo + 1: lo + 1 + m, :], w4b_ref[2],
                        preferred_element_type=f32)
              + b4t_ref[...])
        mw = jnp.concatenate(
            [jnp.maximum(y4[:, 64 * a: 64 * a + 32], y4[:, 64 * a + 32: 64 * a + 64])
             for a in range(4)], axis=1)
        for s in range(4):
            mcat.append(mw[s * _SH: s * _SH + 32, :])
    mcat = jnp.concatenate(mcat, axis=1)
    p4a = jnp.maximum(
        jnp.dot(se16e_ref[...], mcat, preferred_element_type=f32),
        jnp.dot(se16o_ref[...], mcat, preferred_element_type=f32))
    p4a = p4a.astype(bf16)
    for s in range(_BB):
        buf5[_H0 + s * _SH5: _H0 + s * _SH5 + 16, :] = p4a[:, s * 128: s * 128 + 128]

    lo = _H0
    m = _BB * _SH5 - 8
    y5 = (jnp.dot(buf5[lo - 1: lo - 1 + m, :], w5b_ref[0],
                  preferred_element_type=f32)
          + jnp.dot(buf5[lo: lo + m, :], w5b_ref[1],
                    preferred_element_type=f32)
          + jnp.dot(buf5[lo + 1: lo + 1 + m, :], w5b_ref[2],
                    preferred_element_type=f32)
          + b5t_ref[...])
    m5 = jnp.concatenate([jnp.maximum(y5[:, 0:128], y5[:, 128:256]),
                          jnp.maximum(y5[:, 256:384], y5[:, 384:512])],
                         axis=1)
    m5cat = jnp.concatenate(
        [m5[s * _SH5: s * _SH5 + 16, :] for s in range(_BB)], axis=1)
    p5a = jnp.maximum(
        jnp.dot(se8e_ref[...], m5cat, preferred_element_type=f32),
        jnp.dot(se8o_ref[...], m5cat, preferred_element_type=f32))
    p5a = jnp.maximum(p5a, 0.0)
    for s in range(_BB):
        for w2 in range(2):
            c0 = s * 256 + w2 * 128
            out_ref[s, w2 * 8: w2 * 8 + 8, :] = p5a[:, c0: c0 + 128]


def _fc_head_kernel(x_ref, w1_ref, b1_ref, w2_ref, b2_ref, o_ref):
    h = jnp.dot(x_ref[...], w1_ref[...], preferred_element_type=jnp.float32)
    h = h + b1_ref[...]
    y = jnp.dot(h, w2_ref[...], preferred_element_type=jnp.float32)
    o_ref[...] = y + b2_ref[...]


def _conv_features(xr, wfm, b2t, w3b, b3t, w4b, b4t, w5b, b5t,
                   se16e, se16o, se8e, se8o):
    nb = xr.shape[0]
    return pl.pallas_call(
        _conv_kernel,
        out_shape=jax.ShapeDtypeStruct((nb * _BB, 16, 128), jnp.float32),
        grid=(nb,),
        in_specs=[
            pl.BlockSpec((1, 8, _BB * 256), lambda i: (i, 0, 0)),
            pl.BlockSpec((32, 8), lambda i: (0, 0)),
            pl.BlockSpec((32, _BB * 256), lambda i: (0, 0)),
            pl.BlockSpec((3, 256, 512), lambda i: (0, 0, 0)),
            pl.BlockSpec((1, 512), lambda i: (0, 0)),
            pl.BlockSpec((3, 512, 256), lambda i: (0, 0, 0)),
            pl.BlockSpec((1, 256), lambda i: (0, 0)),
            pl.BlockSpec((3, 128, 512), lambda i: (0, 0, 0)),
            pl.BlockSpec((1, 512), lambda i: (0, 0)),
            pl.BlockSpec((16, 32), lambda i: (0, 0)),
            pl.BlockSpec((16, 32), lambda i: (0, 0)),
            pl.BlockSpec((8, 16), lambda i: (0, 0)),
            pl.BlockSpec((8, 16), lambda i: (0, 0)),
        ],
        out_specs=pl.BlockSpec((_BB, 16, 128), lambda i: (i, 0, 0)),
        scratch_shapes=[
            pltpu.VMEM((_NR, 256), jnp.bfloat16),
            pltpu.VMEM((_NR, 512), jnp.bfloat16),
            pltpu.VMEM((_NR5, 128), jnp.bfloat16),
        ],
        compiler_params=pltpu.CompilerParams(dimension_semantics=("parallel",)),
    )(xr, wfm, b2t, w3b, b3t, w4b, b4t, w5b, b5t, se16e, se16o, se8e, se8o)


def _fc_head(person, w1t, b1f, w2p, b2f):
    Bp = person.shape[0]
    bm = next(d for d in (256, 128, 64, 32, 16, 8) if Bp % d == 0)
    return pl.pallas_call(
        _fc_head_kernel,
        out_shape=jax.ShapeDtypeStruct((Bp, 128), jnp.float32),
        grid=(Bp // bm,),
        in_specs=[
            pl.BlockSpec((bm, _FEAT), lambda i: (i, 0)),
            pl.BlockSpec((_FEAT, 256), lambda i: (0, 0)),
            pl.BlockSpec((1, 256), lambda i: (0, 0)),
            pl.BlockSpec((256, 128), lambda i: (0, 0)),
            pl.BlockSpec((1, 128), lambda i: (0, 0)),
        ],
        out_specs=pl.BlockSpec((bm, 128), lambda i: (i, 0)),
        compiler_params=pltpu.CompilerParams(dimension_semantics=("parallel",)),
    )(person, w1t, b1f, w2p, b2f)


def _tridiag(wt, cin, cout, nw):
    f32 = jnp.float32
    out = jnp.zeros((nw * cin, nw * cout), f32)
    ii = jnp.arange(nw)
    for t in range(3):
        e = ((ii[:, None] - ii[None, :]) == (t - 1)).astype(f32)
        out = out + jnp.kron(e, wt[t].astype(f32))
    return out.astype(jnp.bfloat16)


@jax.jit
def _forward(X, wfa, b2m, w3, b3, w4, b4, w5, b5,
             se16e, se16o, se8e, se8o, w1t, b1f, w2p, b2f):
    f32 = jnp.float32
    x = X.reshape(-1, 2, _NUM_JOINTS, _NUM_ACTORS).astype(f32)
    B = x.shape[0]
    Bp = ((B + _BB - 1) // _BB) * _BB
    nb = Bp // _BB

    xpad = jnp.pad(x, ((0, Bp - B), (0, 0), (1, 8), (0, 0)))
    taps = [xpad[:, kind, kh: kh + 32, :].transpose(0, 2, 1)
            for kind in range(2) for kh in range(3)]
    xr = jnp.stack(taps, axis=1)
    xr = xr.reshape(nb, _BB, 6, 256).transpose(0, 2, 1, 3).reshape(nb, 6, _BB * 256)
    xr = jnp.pad(xr, ((0, 0), (0, 2), (0, 0))).astype(jnp.bfloat16)

    wfm = jnp.pad(jnp.transpose(wfa[..., 0], (2, 1, 0)).reshape(32, 6),
                  ((0, 0), (0, 2))).astype(jnp.bfloat16)
    b2t = jnp.tile(b2m, (1, _BB * 8))
    w3b = jnp.stack([_tridiag(w3[:, kh * 32: kh * 32 + 32, :], 32, 64, 8)
                     for kh in range(3)])
    w4b = jnp.stack([_tridiag(w4[:, kh * 64: kh * 64 + 64, :], 64, 32, 8)
                     for kh in range(3)])
    w5b = jnp.stack([_tridiag(w5[:, kh * 32: kh * 32 + 32, :], 32, 128, 4)
                     for kh in range(3)])
    b3t = jnp.tile(b3, (1, 8))
    b4t = jnp.tile(b4, (1, 8))
    b5t = jnp.tile(b5, (1, 4))

    feats = _conv_features(xr, wfm, b2t, w3b, b3t, w4b, b4t, w5b, b5t,
                           se16e, se16o, se8e, se8o)
    person = feats.reshape(Bp, _FEAT)
    out = _fc_head(person, w1t, b1f, w2p, b2f)
    return out[:B, :_NUM_CLASSES]


def kernel(X, wfa, b2m, w3, b3, w4, b4, w5, b5,
           se16e, se16o, se8e, se8o, w1t, b1f, w2p, b2f):
    return _forward(X, wfa, b2m, w3, b3, w4, b4, w5, b5,
                    se16e, se16o, se8e, se8o, w1t, b1f, w2p, b2f)


# speedup vs baseline: 4.6547x; 1.3361x over previous
import jax
import jax.numpy as jnp
from jax.experimental import pallas as pl
from jax.experimental.pallas import tpu as pltpu

_NUM_JOINTS = 25
_NUM_ACTORS = 8
_NUM_CLASSES = 6
_FEAT = 2048

_BB = 16
_SH = 40
_SH5 = 24
_H0 = 8
_NR = _H0 + _BB * _SH + 8
_NR5 = _H0 + _BB * _SH5 + 8


def _conv_kernel(xr_ref, wfm_ref, b2t_ref, w3b_ref, b3t_ref, w4b_ref, b4t_ref,
                 w5b_ref, b5t_ref, se16e_ref, se16o_ref, se8e_ref, se8o_ref,
                 out_ref, buf3, buf4, buf5):
    f32 = jnp.float32
    bf16 = jnp.bfloat16

    for buf, ss, nv in ((buf3, _SH, 32), (buf4, _SH, 32), (buf5, _SH5, 16)):
        buf[0:_H0, :] = jnp.zeros((_H0, buf.shape[1]), bf16)
        for s in range(_BB):
            r = _H0 + s * ss + nv
            buf[r: r + 8, :] = jnp.zeros((8, buf.shape[1]), bf16)

    for g in range(_BB // 8):
        pa = jnp.dot(wfm_ref[...], xr_ref[0, :, g * 2048: g * 2048 + 2048],
                     preferred_element_type=f32)
        pa = (pa + b2t_ref[:, g * 2048: g * 2048 + 2048]).astype(bf16)
        for s0 in range(8):
            s = g * 8 + s0
            buf3[_H0 + s * _SH: _H0 + s * _SH + 32, :] = pa[:, s0 * 256: s0 * 256 + 256]

    for c in range(_BB // 4):
        lo = _H0 + c * 4 * _SH
        m = 4 * _SH - 8
        y3 = (jnp.dot(buf3[lo - 1: lo - 1 + m, :], w3b_ref[0],
                      preferred_element_type=f32)
              + jnp.dot(buf3[lo: lo + m, :], w3b_ref[1],
                        preferred_element_type=f32)
              + jnp.dot(buf3[lo + 1: lo + 1 + m, :], w3b_ref[2],
                        preferred_element_type=f32)
              + b3t_ref[...]).astype(bf16)
        for s in range(4):
            buf4[lo + s * _SH: lo + s * _SH + 32, :] = y3[s * _SH: s * _SH + 32, :]

    mcat = []
    for c in range(_BB // 4):
        lo = _H0 + c * 4 * _SH
        m = 4 * _SH - 8
        y4 = (jnp.dot(buf4[lo - 1: lo - 1 + m, :], w4b_ref[0],
                      preferred_element_type=f32)
              + jnp.dot(buf4[lo: lo + m, :], w4b_ref[1],
                        preferred_element_type=f32)
              + jnp.dot(buf4[lo + 1: lo + 1 + m, :], w4b_ref[2],
                        preferred_element_type=f32)
              + b4t_ref[...])
        mw = jnp.concatenate(
            [jnp.maximum(y4[:, 64 * a: 64 * a + 32], y4[:, 64 * a + 32: 64 * a + 64])
             for a in range(4)], axis=1)
        for s in range(4):
            mcat.append(mw[s * _SH: s * _SH + 32, :])
    mcat = jnp.concatenate(mcat, axis=1)
    p4a = jnp.maximum(
        jnp.dot(se16e_ref[...], mcat, preferred_element_type=f32),
        jnp.dot(se16o_ref[...], mcat, preferred_element_type=f32))
    p4a = p4a.astype(bf16)
    for s in range(_BB):
        buf5[_H0 + s * _SH5: _H0 + s * _SH5 + 16, :] = p4a[:, s * 128: s * 128 + 128]

    m5cat = []
    for g in range(_BB // 8):
        lo = _H0 + g * 8 * _SH5
        m = 8 * _SH5 - 8
        y5 = (jnp.dot(buf5[lo - 1: lo - 1 + m, :], w5b_ref[0],
                      preferred_element_type=f32)
              + jnp.dot(buf5[lo: lo + m, :], w5b_ref[1],
                        preferred_element_type=f32)
              + jnp.dot(buf5[lo + 1: lo + 1 + m, :], w5b_ref[2],
                        preferred_element_type=f32)
              + b5t_ref[...])
        m5 = jnp.concatenate([jnp.maximum(y5[:, 0:128], y5[:, 128:256]),
                              jnp.maximum(y5[:, 256:384], y5[:, 384:512])],
                             axis=1)
        m5cat.extend(m5[s * _SH5: s * _SH5 + 16, :] for s in range(8))
    m5cat = jnp.concatenate(m5cat, axis=1)
    p5a = jnp.maximum(
        jnp.dot(se8e_ref[...], m5cat, preferred_element_type=f32),
        jnp.dot(se8o_ref[...], m5cat, preferred_element_type=f32))
    p5a = jnp.maximum(p5a, 0.0)
    for s in range(_BB):
        for w2 in range(2):
            c0 = s * 256 + w2 * 128
            out_ref[s, w2 * 8: w2 * 8 + 8, :] = p5a[:, c0: c0 + 128]


def _fc_head_kernel(x_ref, w1_ref, b1_ref, w2_ref, b2_ref, o_ref):
    h = jnp.dot(x_ref[...], w1_ref[...], preferred_element_type=jnp.float32)
    h = h + b1_ref[...]
    y = jnp.dot(h, w2_ref[...], preferred_element_type=jnp.float32)
    o_ref[...] = y + b2_ref[...]


def _conv_features(xr, wfm, b2t, w3b, b3t, w4b, b4t, w5b, b5t,
                   se16e, se16o, se8e, se8o):
    nb = xr.shape[0]
    return pl.pallas_call(
        _conv_kernel,
        out_shape=jax.ShapeDtypeStruct((nb * _BB, 16, 128), jnp.float32),
        grid=(nb,),
        in_specs=[
            pl.BlockSpec((1, 8, _BB * 256), lambda i: (i, 0, 0)),
            pl.BlockSpec((32, 8), lambda i: (0, 0)),
            pl.BlockSpec((32, _BB * 256), lambda i: (0, 0)),
            pl.BlockSpec((3, 256, 512), lambda i: (0, 0, 0)),
            pl.BlockSpec((1, 512), lambda i: (0, 0)),
            pl.BlockSpec((3, 512, 256), lambda i: (0, 0, 0)),
            pl.BlockSpec((1, 256), lambda i: (0, 0)),
            pl.BlockSpec((3, 128, 512), lambda i: (0, 0, 0)),
            pl.BlockSpec((1, 512), lambda i: (0, 0)),
            pl.BlockSpec((16, 32), lambda i: (0, 0)),
            pl.BlockSpec((16, 32), lambda i: (0, 0)),
            pl.BlockSpec((8, 16), lambda i: (0, 0)),
            pl.BlockSpec((8, 16), lambda i: (0, 0)),
        ],
        out_specs=pl.BlockSpec((_BB, 16, 128), lambda i: (i, 0, 0)),
        scratch_shapes=[
            pltpu.VMEM((_NR, 256), jnp.bfloat16),
            pltpu.VMEM((_NR, 512), jnp.bfloat16),
            pltpu.VMEM((_NR5, 128), jnp.bfloat16),
        ],
        compiler_params=pltpu.CompilerParams(dimension_semantics=("parallel",)),
    )(xr, wfm, b2t, w3b, b3t, w4b, b4t, w5b, b5t, se16e, se16o, se8e, se8o)


def _fc_head(person, w1t, b1f, w2p, b2f):
    Bp = person.shape[0]
    bm = next(d for d in (256, 128, 64, 32, 16, 8) if Bp % d == 0)
    return pl.pallas_call(
        _fc_head_kernel,
        out_shape=jax.ShapeDtypeStruct((Bp, 128), jnp.float32),
        grid=(Bp // bm,),
        in_specs=[
            pl.BlockSpec((bm, _FEAT), lambda i: (i, 0)),
            pl.BlockSpec((_FEAT, 256), lambda i: (0, 0)),
            pl.BlockSpec((1, 256), lambda i: (0, 0)),
            pl.BlockSpec((256, 128), lambda i: (0, 0)),
            pl.BlockSpec((1, 128), lambda i: (0, 0)),
        ],
        out_specs=pl.BlockSpec((bm, 128), lambda i: (i, 0)),
        compiler_params=pltpu.CompilerParams(dimension_semantics=("parallel",)),
    )(person, w1t, b1f, w2p, b2f)


def _tridiag(wt, cin, cout, nw):
    f32 = jnp.float32
    out = jnp.zeros((nw * cin, nw * cout), f32)
    ii = jnp.arange(nw)
    for t in range(3):
        e = ((ii[:, None] - ii[None, :]) == (t - 1)).astype(f32)
        out = out + jnp.kron(e, wt[t].astype(f32))
    return out.astype(jnp.bfloat16)


@jax.jit
def _forward(X, wfa, b2m, w3, b3, w4, b4, w5, b5,
             se16e, se16o, se8e, se8o, w1t, b1f, w2p, b2f):
    f32 = jnp.float32
    x = X.reshape(-1, 2, _NUM_JOINTS, _NUM_ACTORS).astype(f32)
    B = x.shape[0]
    Bp = ((B + _BB - 1) // _BB) * _BB
    nb = Bp // _BB

    xpad = jnp.pad(x, ((0, Bp - B), (0, 0), (1, 8), (0, 0)))
    taps = [xpad[:, kind, kh: kh + 32, :].transpose(0, 2, 1)
            for kind in range(2) for kh in range(3)]
    xr = jnp.stack(taps, axis=1)
    xr = xr.reshape(nb, _BB, 6, 256).transpose(0, 2, 1, 3).reshape(nb, 6, _BB * 256)
    xr = jnp.pad(xr, ((0, 0), (0, 2), (0, 0))).astype(jnp.bfloat16)

    wfm = jnp.pad(jnp.transpose(wfa[..., 0], (2, 1, 0)).reshape(32, 6),
                  ((0, 0), (0, 2))).astype(jnp.bfloat16)
    b2t = jnp.tile(b2m, (1, _BB * 8))
    w3b = jnp.stack([_tridiag(w3[:, kh * 32: kh * 32 + 32, :], 32, 64, 8)
                     for kh in range(3)])
    w4b = jnp.stack([_tridiag(w4[:, kh * 64: kh * 64 + 64, :], 64, 32, 8)
                     for kh in range(3)])
    w5b = jnp.stack([_tridiag(w5[:, kh * 32: kh * 32 + 32, :], 32, 128, 4)
                     for kh in range(3)])
    b3t = jnp.tile(b3, (1, 8))
    b4t = jnp.tile(b4, (1, 8))
    b5t = jnp.tile(b5, (1, 4))

    feats = _conv_features(xr, wfm, b2t, w3b, b3t, w4b, b4t, w5b, b5t,
                           se16e, se16o, se8e, se8o)
    person = feats.reshape(Bp, _FEAT)
    out = _fc_head(person, w1t, b1f, w2p, b2f)
    return out[:B, :_NUM_CLASSES]


def kernel(X, wfa, b2m, w3, b3, w4, b4, w5, b5,
           se16e, se16o, se8e, se8o, w1t, b1f, w2p, b2f):
    return _forward(X, wfa, b2m, w3, b3, w4, b4, w5, b5,
                    se16e, se16o, se8e, se8o, w1t, b1f, w2p, b2f)


# speedup vs baseline: 5.3509x; 1.1496x over previous
import jax
import jax.numpy as jnp
from jax.experimental import pallas as pl
from jax.experimental.pallas import tpu as pltpu

_NUM_JOINTS = 25
_NUM_ACTORS = 8
_NUM_CLASSES = 6
_FEAT = 2048

_BB = 32
_SH = 40
_SH5 = 24
_H0 = 8
_NR = _H0 + _BB * _SH + 8
_NR5 = _H0 + _BB * _SH5 + 8


def _conv_kernel(xr_ref, wfm_ref, b2t_ref, w3b_ref, b3t_ref, w4b_ref, b4t_ref,
                 w5b_ref, b5t_ref, se16e_ref, se16o_ref, se8e_ref, se8o_ref,
                 out_ref, buf3, buf4, buf5):
    f32 = jnp.float32
    bf16 = jnp.bfloat16

    for buf, ss, nv in ((buf3, _SH, 32), (buf4, _SH, 32), (buf5, _SH5, 16)):
        buf[0:_H0, :] = jnp.zeros((_H0, buf.shape[1]), bf16)
        for s in range(_BB):
            r = _H0 + s * ss + nv
            buf[r: r + 8, :] = jnp.zeros((8, buf.shape[1]), bf16)

    for g in range(_BB // 8):
        pa = jnp.dot(wfm_ref[...], xr_ref[0, :, g * 2048: g * 2048 + 2048],
                     preferred_element_type=f32)
        pa = (pa + b2t_ref[:, g * 2048: g * 2048 + 2048]).astype(bf16)
        for s0 in range(8):
            s = g * 8 + s0
            buf3[_H0 + s * _SH: _H0 + s * _SH + 32, :] = pa[:, s0 * 256: s0 * 256 + 256]

    for c in range(_BB // 4):
        lo = _H0 + c * 4 * _SH
        m = 4 * _SH - 8
        y3 = (jnp.dot(buf3[lo - 1: lo - 1 + m, :], w3b_ref[0],
                      preferred_element_type=f32)
              + jnp.dot(buf3[lo: lo + m, :], w3b_ref[1],
                        preferred_element_type=f32)
              + jnp.dot(buf3[lo + 1: lo + 1 + m, :], w3b_ref[2],
                        preferred_element_type=f32)
              + b3t_ref[...]).astype(bf16)
        for s in range(4):
            buf4[lo + s * _SH: lo + s * _SH + 32, :] = y3[s * _SH: s * _SH + 32, :]

    mcat = []
    for c in range(_BB // 4):
        lo = _H0 + c * 4 * _SH
        m = 4 * _SH - 8
        y4 = (jnp.dot(buf4[lo - 1: lo - 1 + m, :], w4b_ref[0],
                      preferred_element_type=f32)
              + jnp.dot(buf4[lo: lo + m, :], w4b_ref[1],
                        preferred_element_type=f32)
              + jnp.dot(buf4[lo + 1: lo + 1 + m, :], w4b_ref[2],
                        preferred_element_type=f32)
              + b4t_ref[...])
        mw = jnp.concatenate(
            [jnp.maximum(y4[:, 64 * a: 64 * a + 32], y4[:, 64 * a + 32: 64 * a + 64])
             for a in range(4)], axis=1)
        for s in range(4):
            mcat.append(mw[s * _SH: s * _SH + 32, :])
    mcat = jnp.concatenate(mcat, axis=1)
    p4a = jnp.maximum(
        jnp.dot(se16e_ref[...], mcat, preferred_element_type=f32),
        jnp.dot(se16o_ref[...], mcat, preferred_element_type=f32))
    p4a = p4a.astype(bf16)
    for s in range(_BB):
        buf5[_H0 + s * _SH5: _H0 + s * _SH5 + 16, :] = p4a[:, s * 128: s * 128 + 128]

    m5cat = []
    for g in range(_BB // 8):
        lo = _H0 + g * 8 * _SH5
        m = 8 * _SH5 - 8
        y5 = (jnp.dot(buf5[lo - 1: lo - 1 + m, :], w5b_ref[0],
                      preferred_element_type=f32)
              + jnp.dot(buf5[lo: lo + m, :], w5b_ref[1],
                        preferred_element_type=f32)
              + jnp.dot(buf5[lo + 1: lo + 1 + m, :], w5b_ref[2],
                        preferred_element_type=f32)
              + b5t_ref[...])
        m5 = jnp.concatenate([jnp.maximum(y5[:, 0:128], y5[:, 128:256]),
                              jnp.maximum(y5[:, 256:384], y5[:, 384:512])],
                             axis=1)
        m5cat.extend(m5[s * _SH5: s * _SH5 + 16, :] for s in range(8))
    m5cat = jnp.concatenate(m5cat, axis=1)
    p5a = jnp.maximum(
        jnp.dot(se8e_ref[...], m5cat, preferred_element_type=f32),
        jnp.dot(se8o_ref[...], m5cat, preferred_element_type=f32))
    p5a = jnp.maximum(p5a, 0.0)
    for s in range(_BB):
        for w2 in range(2):
            c0 = s * 256 + w2 * 128
            out_ref[s, w2 * 8: w2 * 8 + 8, :] = p5a[:, c0: c0 + 128]


def _fc_head_kernel(x_ref, w1_ref, b1_ref, w2_ref, b2_ref, o_ref):
    h = jnp.dot(x_ref[...], w1_ref[...], preferred_element_type=jnp.float32)
    h = h + b1_ref[...]
    y = jnp.dot(h, w2_ref[...], preferred_element_type=jnp.float32)
    o_ref[...] = y + b2_ref[...]


def _conv_features(xr, wfm, b2t, w3b, b3t, w4b, b4t, w5b, b5t,
                   se16e, se16o, se8e, se8o):
    nb = xr.shape[0]
    return pl.pallas_call(
        _conv_kernel,
        out_shape=jax.ShapeDtypeStruct((nb * _BB, 16, 128), jnp.float32),
        grid=(nb,),
        in_specs=[
            pl.BlockSpec((1, 8, _BB * 256), lambda i: (i, 0, 0)),
            pl.BlockSpec((32, 8), lambda i: (0, 0)),
            pl.BlockSpec((32, _BB * 256), lambda i: (0, 0)),
            pl.BlockSpec((3, 256, 512), lambda i: (0, 0, 0)),
            pl.BlockSpec((1, 512), lambda i: (0, 0)),
            pl.BlockSpec((3, 512, 256), lambda i: (0, 0, 0)),
            pl.BlockSpec((1, 256), lambda i: (0, 0)),
            pl.BlockSpec((3, 128, 512), lambda i: (0, 0, 0)),
            pl.BlockSpec((1, 512), lambda i: (0, 0)),
            pl.BlockSpec((16, 32), lambda i: (0, 0)),
            pl.BlockSpec((16, 32), lambda i: (0, 0)),
            pl.BlockSpec((8, 16), lambda i: (0, 0)),
            pl.BlockSpec((8, 16), lambda i: (0, 0)),
        ],
        out_specs=pl.BlockSpec((_BB, 16, 128), lambda i: (i, 0, 0)),
        scratch_shapes=[
            pltpu.VMEM((_NR, 256), jnp.bfloat16),
            pltpu.VMEM((_NR, 512), jnp.bfloat16),
            pltpu.VMEM((_NR5, 128), jnp.bfloat16),
        ],
        compiler_params=pltpu.CompilerParams(dimension_semantics=("parallel",)),
    )(xr, wfm, b2t, w3b, b3t, w4b, b4t, w5b, b5t, se16e, se16o, se8e, se8o)


def _fc_head(person, w1t, b1f, w2p, b2f):
    Bp = person.shape[0]
    bm = next(d for d in (256, 128, 64, 32, 16, 8) if Bp % d == 0)
    return pl.pallas_call(
        _fc_head_kernel,
        out_shape=jax.ShapeDtypeStruct((Bp, 128), jnp.float32),
        grid=(Bp // bm,),
        in_specs=[
            pl.BlockSpec((bm, _FEAT), lambda i: (i, 0)),
            pl.BlockSpec((_FEAT, 256), lambda i: (0, 0)),
            pl.BlockSpec((1, 256), lambda i: (0, 0)),
            pl.BlockSpec((256, 128), lambda i: (0, 0)),
            pl.BlockSpec((1, 128), lambda i: (0, 0)),
        ],
        out_specs=pl.BlockSpec((bm, 128), lambda i: (i, 0)),
        compiler_params=pltpu.CompilerParams(dimension_semantics=("parallel",)),
    )(person, w1t, b1f, w2p, b2f)


def _tridiag(wt, cin, cout, nw):
    f32 = jnp.float32
    out = jnp.zeros((nw * cin, nw * cout), f32)
    ii = jnp.arange(nw)
    for t in range(3):
        e = ((ii[:, None] - ii[None, :]) == (t - 1)).astype(f32)
        out = out + jnp.kron(e, wt[t].astype(f32))
    return out.astype(jnp.bfloat16)


@jax.jit
def _forward(X, wfa, b2m, w3, b3, w4, b4, w5, b5,
             se16e, se16o, se8e, se8o, w1t, b1f, w2p, b2f):
    f32 = jnp.float32
    x = X.reshape(-1, 2, _NUM_JOINTS, _NUM_ACTORS).astype(f32)
    B = x.shape[0]
    Bp = ((B + _BB - 1) // _BB) * _BB
    nb = Bp // _BB

    xpad = jnp.pad(x, ((0, Bp - B), (0, 0), (1, 8), (0, 0)))
    taps = [xpad[:, kind, kh: kh + 32, :].transpose(0, 2, 1)
            for kind in range(2) for kh in range(3)]
    xr = jnp.stack(taps, axis=1)
    xr = xr.reshape(nb, _BB, 6, 256).transpose(0, 2, 1, 3).reshape(nb, 6, _BB * 256)
    xr = jnp.pad(xr, ((0, 0), (0, 2), (0, 0))).astype(jnp.bfloat16)

    wfm = jnp.pad(jnp.transpose(wfa[..., 0], (2, 1, 0)).reshape(32, 6),
                  ((0, 0), (0, 2))).astype(jnp.bfloat16)
    b2t = jnp.tile(b2m, (1, _BB * 8))
    w3b = jnp.stack([_tridiag(w3[:, kh * 32: kh * 32 + 32, :], 32, 64, 8)
                     for kh in range(3)])
    w4b = jnp.stack([_tridiag(w4[:, kh * 64: kh * 64 + 64, :], 64, 32, 8)
                     for kh in range(3)])
    w5b = jnp.stack([_tridiag(w5[:, kh * 32: kh * 32 + 32, :], 32, 128, 4)
                     for kh in range(3)])
    b3t = jnp.tile(b3, (1, 8))
    b4t = jnp.tile(b4, (1, 8))
    b5t = jnp.tile(b5, (1, 4))

    feats = _conv_features(xr, wfm, b2t, w3b, b3t, w4b, b4t, w5b, b5t,
                           se16e, se16o, se8e, se8o)
    person = feats.reshape(Bp, _FEAT)
    out = _fc_head(person, w1t, b1f, w2p, b2f)
    return out[:B, :_NUM_CLASSES]


def kernel(X, wfa, b2m, w3, b3, w4, b4, w5, b5,
           se16e, se16o, se8e, se8o, w1t, b1f, w2p, b2f):
    return _forward(X, wfa, b2m, w3, b3, w4, b4, w5, b5,
                    se16e, se16o, se8e, se8o, w1t, b1f, w2p, b2f)


# speedup vs baseline: 6.2428x; 1.1667x over previous
import jax
import jax.numpy as jnp
from jax.experimental import pallas as pl
from jax.experimental.pallas import tpu as pltpu

_NUM_JOINTS = 25
_NUM_ACTORS = 8
_NUM_CLASSES = 6
_FEAT = 2048

_BB = 64
_SH = 40
_SH5 = 24
_H0 = 8
_NR = _H0 + _BB * _SH + 8
_NR5 = _H0 + _BB * _SH5 + 8


def _conv_kernel(xr_ref, wfm_ref, b2t_ref, w3b_ref, b3t_ref, w4b_ref, b4t_ref,
                 w5b_ref, b5t_ref, se16e_ref, se16o_ref, se8e_ref, se8o_ref,
                 out_ref, buf3, buf4, buf5):
    f32 = jnp.float32
    bf16 = jnp.bfloat16

    for buf, ss, nv in ((buf3, _SH, 32), (buf4, _SH, 32), (buf5, _SH5, 16)):
        buf[0:_H0, :] = jnp.zeros((_H0, buf.shape[1]), bf16)
        for s in range(_BB):
            r = _H0 + s * ss + nv
            buf[r: r + 8, :] = jnp.zeros((8, buf.shape[1]), bf16)

    for g in range(_BB // 8):
        pa = jnp.dot(wfm_ref[...], xr_ref[0, :, g * 2048: g * 2048 + 2048],
                     preferred_element_type=f32)
        pa = (pa + b2t_ref[:, g * 2048: g * 2048 + 2048]).astype(bf16)
        for s0 in range(8):
            s = g * 8 + s0
            buf3[_H0 + s * _SH: _H0 + s * _SH + 32, :] = pa[:, s0 * 256: s0 * 256 + 256]

    for c in range(_BB // 4):
        lo = _H0 + c * 4 * _SH
        m = 4 * _SH - 8
        y3 = (jnp.dot(buf3[lo - 1: lo - 1 + m, :], w3b_ref[0],
                      preferred_element_type=f32)
              + jnp.dot(buf3[lo: lo + m, :], w3b_ref[1],
                        preferred_element_type=f32)
              + jnp.dot(buf3[lo + 1: lo + 1 + m, :], w3b_ref[2],
                        preferred_element_type=f32)
              + b3t_ref[...]).astype(bf16)
        for s in range(4):
            buf4[lo + s * _SH: lo + s * _SH + 32, :] = y3[s * _SH: s * _SH + 32, :]

    mcat = []
    for c in range(_BB // 4):
        lo = _H0 + c * 4 * _SH
        m = 4 * _SH - 8
        y4 = (jnp.dot(buf4[lo - 1: lo - 1 + m, :], w4b_ref[0],
                      preferred_element_type=f32)
              + jnp.dot(buf4[lo: lo + m, :], w4b_ref[1],
                        preferred_element_type=f32)
              + jnp.dot(buf4[lo + 1: lo + 1 + m, :], w4b_ref[2],
                        preferred_element_type=f32)
              + b4t_ref[...])
        mw = jnp.concatenate(
            [jnp.maximum(y4[:, 64 * a: 64 * a + 32], y4[:, 64 * a + 32: 64 * a + 64])
             for a in range(4)], axis=1)
        for s in range(4):
            mcat.append(mw[s * _SH: s * _SH + 32, :])
    mcat = jnp.concatenate(mcat, axis=1)
    p4a = jnp.maximum(
        jnp.dot(se16e_ref[...], mcat, preferred_element_type=f32),
        jnp.dot(se16o_ref[...], mcat, preferred_element_type=f32))
    p4a = p4a.astype(bf16)
    for s in range(_BB):
        buf5[_H0 + s * _SH5: _H0 + s * _SH5 + 16, :] = p4a[:, s * 128: s * 128 + 128]

    m5cat = []
    for g in range(_BB // 8):
        lo = _H0 + g * 8 * _SH5
        m = 8 * _SH5 - 8
        y5 = (jnp.dot(buf5[lo - 1: lo - 1 + m, :], w5b_ref[0],
                      preferred_element_type=f32)
              + jnp.dot(buf5[lo: lo + m, :], w5b_ref[1],
                        preferred_element_type=f32)
              + jnp.dot(buf5[lo + 1: lo + 1 + m, :], w5b_ref[2],
                        preferred_element_type=f32)
              + b5t_ref[...])
        m5 = jnp.concatenate([jnp.maximum(y5[:, 0:128], y5[:, 128:256]),
                              jnp.maximum(y5[:, 256:384], y5[:, 384:512])],
                             axis=1)
        m5cat.extend(m5[s * _SH5: s * _SH5 + 16, :] for s in range(8))
    m5cat = jnp.concatenate(m5cat, axis=1)
    p5a = jnp.maximum(
        jnp.dot(se8e_ref[...], m5cat, preferred_element_type=f32),
        jnp.dot(se8o_ref[...], m5cat, preferred_element_type=f32))
    p5a = jnp.maximum(p5a, 0.0)
    for s in range(_BB):
        for w2 in range(2):
            c0 = s * 256 + w2 * 128
            out_ref[s, w2 * 8: w2 * 8 + 8, :] = p5a[:, c0: c0 + 128]


def _fc_head_kernel(x_ref, w1_ref, b1_ref, w2_ref, b2_ref, o_ref):
    h = jnp.dot(x_ref[...], w1_ref[...], preferred_element_type=jnp.float32)
    h = h + b1_ref[...]
    y = jnp.dot(h, w2_ref[...], preferred_element_type=jnp.float32)
    o_ref[...] = y + b2_ref[...]


def _conv_features(xr, wfm, b2t, w3b, b3t, w4b, b4t, w5b, b5t,
                   se16e, se16o, se8e, se8o):
    nb = xr.shape[0]
    return pl.pallas_call(
        _conv_kernel,
        out_shape=jax.ShapeDtypeStruct((nb * _BB, 16, 128), jnp.float32),
        grid=(nb,),
        in_specs=[
            pl.BlockSpec((1, 8, _BB * 256), lambda i: (i, 0, 0)),
            pl.BlockSpec((32, 8), lambda i: (0, 0)),
            pl.BlockSpec((32, _BB * 256), lambda i: (0, 0)),
            pl.BlockSpec((3, 256, 512), lambda i: (0, 0, 0)),
            pl.BlockSpec((1, 512), lambda i: (0, 0)),
            pl.BlockSpec((3, 512, 256), lambda i: (0, 0, 0)),
            pl.BlockSpec((1, 256), lambda i: (0, 0)),
            pl.BlockSpec((3, 128, 512), lambda i: (0, 0, 0)),
            pl.BlockSpec((1, 512), lambda i: (0, 0)),
            pl.BlockSpec((16, 32), lambda i: (0, 0)),
            pl.BlockSpec((16, 32), lambda i: (0, 0)),
            pl.BlockSpec((8, 16), lambda i: (0, 0)),
            pl.BlockSpec((8, 16), lambda i: (0, 0)),
        ],
        out_specs=pl.BlockSpec((_BB, 16, 128), lambda i: (i, 0, 0)),
        scratch_shapes=[
            pltpu.VMEM((_NR, 256), jnp.bfloat16),
            pltpu.VMEM((_NR, 512), jnp.bfloat16),
            pltpu.VMEM((_NR5, 128), jnp.bfloat16),
        ],
        compiler_params=pltpu.CompilerParams(dimension_semantics=("parallel",)),
    )(xr, wfm, b2t, w3b, b3t, w4b, b4t, w5b, b5t, se16e, se16o, se8e, se8o)


def _fc_head(person, w1t, b1f, w2p, b2f):
    Bp = person.shape[0]
    bm = next(d for d in (256, 128, 64, 32, 16, 8) if Bp % d == 0)
    return pl.pallas_call(
        _fc_head_kernel,
        out_shape=jax.ShapeDtypeStruct((Bp, 128), jnp.float32),
        grid=(Bp // bm,),
        in_specs=[
            pl.BlockSpec((bm, _FEAT), lambda i: (i, 0)),
            pl.BlockSpec((_FEAT, 256), lambda i: (0, 0)),
            pl.BlockSpec((1, 256), lambda i: (0, 0)),
            pl.BlockSpec((256, 128), lambda i: (0, 0)),
            pl.BlockSpec((1, 128), lambda i: (0, 0)),
        ],
        out_specs=pl.BlockSpec((bm, 128), lambda i: (i, 0)),
        compiler_params=pltpu.CompilerParams(dimension_semantics=("parallel",)),
    )(person, w1t, b1f, w2p, b2f)


def _tridiag(wt, cin, cout, nw):
    f32 = jnp.float32
    out = jnp.zeros((nw * cin, nw * cout), f32)
    ii = jnp.arange(nw)
    for t in range(3):
        e = ((ii[:, None] - ii[None, :]) == (t - 1)).astype(f32)
        out = out + jnp.kron(e, wt[t].astype(f32))
    return out.astype(jnp.bfloat16)


@jax.jit
def _forward(X, wfa, b2m, w3, b3, w4, b4, w5, b5,
             se16e, se16o, se8e, se8o, w1t, b1f, w2p, b2f):
    f32 = jnp.float32
    x = X.reshape(-1, 2, _NUM_JOINTS, _NUM_ACTORS).astype(f32)
    B = x.shape[0]
    Bp = ((B + _BB - 1) // _BB) * _BB
    nb = Bp // _BB

    xpad = jnp.pad(x, ((0, Bp - B), (0, 0), (1, 8), (0, 0)))
    taps = [xpad[:, kind, kh: kh + 32, :].transpose(0, 2, 1)
            for kind in range(2) for kh in range(3)]
    xr = jnp.stack(taps, axis=1)
    xr = xr.reshape(nb, _BB, 6, 256).transpose(0, 2, 1, 3).reshape(nb, 6, _BB * 256)
    xr = jnp.pad(xr, ((0, 0), (0, 2), (0, 0))).astype(jnp.bfloat16)

    wfm = jnp.pad(jnp.transpose(wfa[..., 0], (2, 1, 0)).reshape(32, 6),
                  ((0, 0), (0, 2))).astype(jnp.bfloat16)
    b2t = jnp.tile(b2m, (1, _BB * 8))
    w3b = jnp.stack([_tridiag(w3[:, kh * 32: kh * 32 + 32, :], 32, 64, 8)
                     for kh in range(3)])
    w4b = jnp.stack([_tridiag(w4[:, kh * 64: kh * 64 + 64, :], 64, 32, 8)
                     for kh in range(3)])
    w5b = jnp.stack([_tridiag(w5[:, kh * 32: kh * 32 + 32, :], 32, 128, 4)
                     for kh in range(3)])
    b3t = jnp.tile(b3, (1, 8))
    b4t = jnp.tile(b4, (1, 8))
    b5t = jnp.tile(b5, (1, 4))

    feats = _conv_features(xr, wfm, b2t, w3b, b3t, w4b, b4t, w5b, b5t,
                           se16e, se16o, se8e, se8o)
    person = feats.reshape(Bp, _FEAT)
    out = _fc_head(person, w1t, b1f, w2p, b2f)
    return out[:B, :_NUM_CLASSES]


def kernel(X, wfa, b2m, w3, b3, w4, b4, w5, b5,
           se16e, se16o, se8e, se8o, w1t, b1f, w2p, b2f):
    return _forward(X, wfa, b2m, w3, b3, w4, b4, w5, b5,
                    se16e, se16o, se8e, se8o, w1t, b1f, w2p, b2f)


# speedup vs baseline: 6.9490x; 1.1131x over previous
import jax
import jax.numpy as jnp
from jax.experimental import pallas as pl
from jax.experimental.pallas import tpu as pltpu

_NUM_JOINTS = 25
_NUM_ACTORS = 8
_NUM_CLASSES = 6
_FEAT = 2048

_BB = 128
_SH = 40
_SH5 = 24
_H0 = 8
_NR = _H0 + _BB * _SH + 8
_NR5 = _H0 + _BB * _SH5 + 8


def _conv_kernel(xr_ref, wfm_ref, b2t_ref, w3b_ref, b3t_ref, w4b_ref, b4t_ref,
                 w5b_ref, b5t_ref, se16e_ref, se16o_ref, se8e_ref, se8o_ref,
                 out_ref, buf3, buf4, buf5):
    f32 = jnp.float32
    bf16 = jnp.bfloat16

    for buf, ss, nv in ((buf3, _SH, 32), (buf4, _SH, 32), (buf5, _SH5, 16)):
        buf[0:_H0, :] = jnp.zeros((_H0, buf.shape[1]), bf16)
        for s in range(_BB):
            r = _H0 + s * ss + nv
            buf[r: r + 8, :] = jnp.zeros((8, buf.shape[1]), bf16)

    for g in range(_BB // 8):
        pa = jnp.dot(wfm_ref[...], xr_ref[0, :, g * 2048: g * 2048 + 2048],
                     preferred_element_type=f32)
        pa = (pa + b2t_ref[:, g * 2048: g * 2048 + 2048]).astype(bf16)
        for s0 in range(8):
            s = g * 8 + s0
            buf3[_H0 + s * _SH: _H0 + s * _SH + 32, :] = pa[:, s0 * 256: s0 * 256 + 256]

    for c in range(_BB // 4):
        lo = _H0 + c * 4 * _SH
        m = 4 * _SH - 8
        y3 = (jnp.dot(buf3[lo - 1: lo - 1 + m, :], w3b_ref[0],
                      preferred_element_type=f32)
              + jnp.dot(buf3[lo: lo + m, :], w3b_ref[1],
                        preferred_element_type=f32)
              + jnp.dot(buf3[lo + 1: lo + 1 + m, :], w3b_ref[2],
                        preferred_element_type=f32)
              + b3t_ref[...]).astype(bf16)
        for s in range(4):
            buf4[lo + s * _SH: lo + s * _SH + 32, :] = y3[s * _SH: s * _SH + 32, :]

    mcat = []
    for c in range(_BB // 4):
        lo = _H0 + c * 4 * _SH
        m = 4 * _SH - 8
        y4 = (jnp.dot(buf4[lo - 1: lo - 1 + m, :], w4b_ref[0],
                      preferred_element_type=f32)
              + jnp.dot(buf4[lo: lo + m, :], w4b_ref[1],
                        preferred_element_type=f32)
              + jnp.dot(buf4[lo + 1: lo + 1 + m, :], w4b_ref[2],
                        preferred_element_type=f32)
              + b4t_ref[...])
        mw = jnp.concatenate(
            [jnp.maximum(y4[:, 64 * a: 64 * a + 32], y4[:, 64 * a + 32: 64 * a + 64])
             for a in range(4)], axis=1)
        for s in range(4):
            mcat.append(mw[s * _SH: s * _SH + 32, :])
    mcat = jnp.concatenate(mcat, axis=1)
    p4a = jnp.maximum(
        jnp.dot(se16e_ref[...], mcat, preferred_element_type=f32),
        jnp.dot(se16o_ref[...], mcat, preferred_element_type=f32))
    p4a = p4a.astype(bf16)
    for s in range(_BB):
        buf5[_H0 + s * _SH5: _H0 + s * _SH5 + 16, :] = p4a[:, s * 128: s * 128 + 128]

    m5cat = []
    for g in range(_BB // 8):
        lo = _H0 + g * 8 * _SH5
        m = 8 * _SH5 - 8
        y5 = (jnp.dot(buf5[lo - 1: lo - 1 + m, :], w5b_ref[0],
                      preferred_element_type=f32)
              + jnp.dot(buf5[lo: lo + m, :], w5b_ref[1],
                        preferred_element_type=f32)
              + jnp.dot(buf5[lo + 1: lo + 1 + m, :], w5b_ref[2],
                        preferred_element_type=f32)
              + b5t_ref[...])
        m5 = jnp.concatenate([jnp.maximum(y5[:, 0:128], y5[:, 128:256]),
                              jnp.maximum(y5[:, 256:384], y5[:, 384:512])],
                             axis=1)
        m5cat.extend(m5[s * _SH5: s * _SH5 + 16, :] for s in range(8))
    m5cat = jnp.concatenate(m5cat, axis=1)
    p5a = jnp.maximum(
        jnp.dot(se8e_ref[...], m5cat, preferred_element_type=f32),
        jnp.dot(se8o_ref[...], m5cat, preferred_element_type=f32))
    p5a = jnp.maximum(p5a, 0.0)
    for s in range(_BB):
        for w2 in range(2):
            c0 = s * 256 + w2 * 128
            out_ref[s, w2 * 8: w2 * 8 + 8, :] = p5a[:, c0: c0 + 128]


def _fc_head_kernel(x_ref, w1_ref, b1_ref, w2_ref, b2_ref, o_ref):
    h = jnp.dot(x_ref[...], w1_ref[...], preferred_element_type=jnp.float32)
    h = h + b1_ref[...]
    y = jnp.dot(h, w2_ref[...], preferred_element_type=jnp.float32)
    o_ref[...] = y + b2_ref[...]


def _conv_features(xr, wfm, b2t, w3b, b3t, w4b, b4t, w5b, b5t,
                   se16e, se16o, se8e, se8o):
    nb = xr.shape[0]
    return pl.pallas_call(
        _conv_kernel,
        out_shape=jax.ShapeDtypeStruct((nb * _BB, 16, 128), jnp.float32),
        grid=(nb,),
        in_specs=[
            pl.BlockSpec((1, 8, _BB * 256), lambda i: (i, 0, 0)),
            pl.BlockSpec((32, 8), lambda i: (0, 0)),
            pl.BlockSpec((32, _BB * 256), lambda i: (0, 0)),
            pl.BlockSpec((3, 256, 512), lambda i: (0, 0, 0)),
            pl.BlockSpec((1, 512), lambda i: (0, 0)),
            pl.BlockSpec((3, 512, 256), lambda i: (0, 0, 0)),
            pl.BlockSpec((1, 256), lambda i: (0, 0)),
            pl.BlockSpec((3, 128, 512), lambda i: (0, 0, 0)),
            pl.BlockSpec((1, 512), lambda i: (0, 0)),
            pl.BlockSpec((16, 32), lambda i: (0, 0)),
            pl.BlockSpec((16, 32), lambda i: (0, 0)),
            pl.BlockSpec((8, 16), lambda i: (0, 0)),
            pl.BlockSpec((8, 16), lambda i: (0, 0)),
        ],
        out_specs=pl.BlockSpec((_BB, 16, 128), lambda i: (i, 0, 0)),
        scratch_shapes=[
            pltpu.VMEM((_NR, 256), jnp.bfloat16),
            pltpu.VMEM((_NR, 512), jnp.bfloat16),
            pltpu.VMEM((_NR5, 128), jnp.bfloat16),
        ],
        compiler_params=pltpu.CompilerParams(dimension_semantics=("parallel",)),
    )(xr, wfm, b2t, w3b, b3t, w4b, b4t, w5b, b5t, se16e, se16o, se8e, se8o)


def _fc_head(person, w1t, b1f, w2p, b2f):
    Bp = person.shape[0]
    bm = next(d for d in (256, 128, 64, 32, 16, 8) if Bp % d == 0)
    return pl.pallas_call(
        _fc_head_kernel,
        out_shape=jax.ShapeDtypeStruct((Bp, 128), jnp.float32),
        grid=(Bp // bm,),
        in_specs=[
            pl.BlockSpec((bm, _FEAT), lambda i: (i, 0)),
            pl.BlockSpec((_FEAT, 256), lambda i: (0, 0)),
            pl.BlockSpec((1, 256), lambda i: (0, 0)),
            pl.BlockSpec((256, 128), lambda i: (0, 0)),
            pl.BlockSpec((1, 128), lambda i: (0, 0)),
        ],
        out_specs=pl.BlockSpec((bm, 128), lambda i: (i, 0)),
        compiler_params=pltpu.CompilerParams(dimension_semantics=("parallel",)),
    )(person, w1t, b1f, w2p, b2f)


def _tridiag(wt, cin, cout, nw):
    f32 = jnp.float32
    out = jnp.zeros((nw * cin, nw * cout), f32)
    ii = jnp.arange(nw)
    for t in range(3):
        e = ((ii[:, None] - ii[None, :]) == (t - 1)).astype(f32)
        out = out + jnp.kron(e, wt[t].astype(f32))
    return out.astype(jnp.bfloat16)


@jax.jit
def _forward(X, wfa, b2m, w3, b3, w4, b4, w5, b5,
             se16e, se16o, se8e, se8o, w1t, b1f, w2p, b2f):
    f32 = jnp.float32
    x = X.reshape(-1, 2, _NUM_JOINTS, _NUM_ACTORS).astype(f32)
    B = x.shape[0]
    Bp = ((B + _BB - 1) // _BB) * _BB
    nb = Bp // _BB

    xpad = jnp.pad(x, ((0, Bp - B), (0, 0), (1, 8), (0, 0)))
    taps = [xpad[:, kind, kh: kh + 32, :].transpose(0, 2, 1)
            for kind in range(2) for kh in range(3)]
    xr = jnp.stack(taps, axis=1)
    xr = xr.reshape(nb, _BB, 6, 256).transpose(0, 2, 1, 3).reshape(nb, 6, _BB * 256)
    xr = jnp.pad(xr, ((0, 0), (0, 2), (0, 0))).astype(jnp.bfloat16)

    wfm = jnp.pad(jnp.transpose(wfa[..., 0], (2, 1, 0)).reshape(32, 6),
                  ((0, 0), (0, 2))).astype(jnp.bfloat16)
    b2t = jnp.tile(b2m, (1, _BB * 8))
    w3b = jnp.stack([_tridiag(w3[:, kh * 32: kh * 32 + 32, :], 32, 64, 8)
                     for kh in range(3)])
    w4b = jnp.stack([_tridiag(w4[:, kh * 64: kh * 64 + 64, :], 64, 32, 8)
                     for kh in range(3)])
    w5b = jnp.stack([_tridiag(w5[:, kh * 32: kh * 32 + 32, :], 32, 128, 4)
                     for kh in range(3)])
    b3t = jnp.tile(b3, (1, 8))
    b4t = jnp.tile(b4, (1, 8))
    b5t = jnp.tile(b5, (1, 4))

    feats = _conv_features(xr, wfm, b2t, w3b, b3t, w4b, b4t, w5b, b5t,
                           se16e, se16o, se8e, se8o)
    person = feats.reshape(Bp, _FEAT)
    out = _fc_head(person, w1t, b1f, w2p, b2f)
    return out[:B, :_NUM_CLASSES]


def kernel(X, wfa, b2m, w3, b3, w4, b4, w5, b5,
           se16e, se16o, se8e, se8o, w1t, b1f, w2p, b2f):
    return _forward(X, wfa, b2m, w3, b3, w4, b4, w5, b5,
                    se16e, se16o, se8e, se8o, w1t, b1f, w2p, b2f)
